# Initial kernel scaffold; baseline (speedup 1.0000x reference)
import jax
import jax.numpy as jnp
from jax.experimental import pallas as pl


def kernel(pillar_features, coords):
    raise NotImplementedError("write your pallas kernel here")


# speedup vs baseline: 1.1228x; 1.1228x over previous
import functools

import jax
import jax.numpy as jnp
from jax import lax
from jax.experimental import pallas as pl
from jax.experimental.pallas import tpu as pltpu
from jax.experimental.pallas import tpu_sc as plsc

H, W = 512, 512
HW = H * W
C = 64
P = 20000

NC, NS = 2, 16
NW = NC * NS
TILE_RANGE = HW // NW
BS = 512
NB = TILE_RANGE // BS
CHUNK = 2000
NCHUNK = P // CHUNK
LANES = 16


def _scatter_body(xs_hbm, ys_hbm, feat_hbm, out_hbm,
                  xbuf, ybuf, list_ref, sub_ref, feat_stage, block_ref, gsem):
    wid = lax.axis_index("s") * NC + lax.axis_index("c")
    r0 = wid * TILE_RANGE
    iota = lax.iota(jnp.int32, LANES)
    zeros16 = jnp.zeros((LANES,), jnp.float32)

    def chunk_body(ck, n):
        pltpu.sync_copy(xs_hbm.at[pl.ds(ck * CHUNK, CHUNK)], xbuf)
        pltpu.sync_copy(ys_hbm.at[pl.ds(ck * CHUNK, CHUNK)], ybuf)

        def vec_body(v, n):
            x = xbuf[pl.ds(v * LANES, LANES)]
            y = ybuf[pl.ds(v * LANES, LANES)]
            off = (y * W + x) - r0
            m = (off >= 0) & (off < TILE_RANGE)
            i_vec = ck * CHUNK + v * LANES + iota
            e = i_vec * TILE_RANGE + off
            mi = m.astype(jnp.int32)
            pos = n + plsc.cumsum(mi) - 1
            plsc.store_scatter(list_ref, [pos], e, mask=m)
            return n + jnp.sum(mi)

        return lax.fori_loop(0, CHUNK // LANES, vec_body, n)

    n = lax.fori_loop(0, NCHUNK, chunk_body, jnp.int32(0))
    nvec = (n + LANES - 1) // LANES

    def block_body(b, _):
        def z_body(c, _):
            def z_inner(k, _):
                block_ref[c, pl.ds(k * LANES, LANES)] = zeros16
                return 0
            return lax.fori_loop(0, BS // LANES, z_inner, 0)
        lax.fori_loop(0, C, z_body, 0)

        def scan_body(v, m):
            e = list_ref[pl.ds(v * LANES, LANES)]
            valid = (v * LANES + iota) < n
            off = e & (TILE_RANGE - 1)
            sel = valid & ((off >> 9) == b)
            seli = sel.astype(jnp.int32)
            pos = m + plsc.cumsum(seli) - 1
            plsc.store_scatter(sub_ref, [pos], e, mask=sel)
            return m + jnp.sum(seli)

        m = lax.fori_loop(0, nvec, scan_body, jnp.int32(0))

        def grp_body(g, _):
            eg = sub_ref[pl.ds(g * LANES, LANES)]
            gvalid = (g * LANES + iota) < m
            idxv = jnp.where(gvalid, eg >> 13, 0)
            pltpu.async_copy(feat_hbm.at[idxv], feat_stage, gsem).wait()
            cnt = jnp.minimum(m - g * LANES, LANES)

            def pt_body(p, _):
                ep = sub_ref[pl.ds(g * LANES + p, LANES)][0]
                j = ep & (BS - 1)
                cols = jnp.full((LANES,), j, jnp.int32)
                for q in range(C // LANES):
                    vec = feat_stage[p, pl.ds(q * LANES, LANES)]
                    rows = iota + q * LANES
                    plsc.store_scatter(block_ref, [rows, cols], vec)
                return 0

            lax.fori_loop(0, cnt, pt_body, 0)
            return 0

        lax.fori_loop(0, (m + LANES - 1) // LANES, grp_body, 0)

        pltpu.sync_copy(block_ref, out_hbm.at[:, pl.ds(r0 + b * BS, BS)])
        return 0

    lax.fori_loop(0, NB, block_body, 0)


@jax.jit
def _scatter(xs, ys, feat):
    mesh = plsc.VectorSubcoreMesh(core_axis_name="c", subcore_axis_name="s",
                                  num_cores=NC, num_subcores=NS)
    return pl.kernel(
        _scatter_body,
        out_type=jax.ShapeDtypeStruct((C, HW), jnp.float32),
        mesh=mesh,
        compiler_params=pltpu.CompilerParams(needs_layout_passes=False, use_tc_tiling_on_sc=False),
        scratch_types=[
            pltpu.VMEM((CHUNK,), jnp.int32),
            pltpu.VMEM((CHUNK,), jnp.int32),
            pltpu.VMEM((P + LANES,), jnp.int32),
            pltpu.VMEM((P + LANES,), jnp.int32),
            pltpu.VMEM((LANES, C), jnp.float32),
            pltpu.VMEM((C, BS), jnp.float32),
            pltpu.SemaphoreType.DMA,
        ],
    )(xs, ys, feat)


def kernel(pillar_features, coords):
    xs = jnp.asarray(coords[:, 0], jnp.int32)
    ys = jnp.asarray(coords[:, 1], jnp.int32)
    canvas = _scatter(xs, ys, pillar_features)
    return canvas.reshape(1, C, H, W)
